# Initial kernel scaffold; baseline (speedup 1.0000x reference)
import jax
import jax.numpy as jnp
from jax.experimental import pallas as pl


def kernel(z, embedding):
    raise NotImplementedError("write your pallas kernel here")


# speedup vs baseline: 1.7709x; 1.7709x over previous
import jax
import jax.numpy as jnp
from jax.experimental import pallas as pl


def _vq_body(z_ref, emb_ref, zq_ref, loss_ref, closs_ref, qloss_ref, idx_ref):
    E = emb_ref[...]
    Z = z_ref[0]
    e_sq = jnp.sum(E * E, axis=1, keepdims=True)
    z_sq = jnp.sum(Z * Z, axis=0, keepdims=True)
    mm = jax.lax.dot_general(
        E, Z, (((1,), (0,)), ((), ())),
        preferred_element_type=jnp.float32)
    d = (z_sq + e_sq) - 2.0 * mm
    dmin = jnp.min(d, axis=0, keepdims=True)
    iota = jax.lax.broadcasted_iota(jnp.int32, d.shape, 0)
    big = jnp.int32(d.shape[0])
    idx = jnp.min(jnp.where(d == dmin, iota, big), axis=0,
                  keepdims=True)
    onehot = (iota == idx).astype(jnp.float32)
    zq_t = jax.lax.dot_general(
        E, onehot, (((0,), (0,)), ((), ())),
        preferred_element_type=jnp.float32)
    zq_ref[0] = zq_t
    diff = zq_t - Z
    sq_t = diff * diff
    sq = sq_t.T
    loss_ref[0] = 1.25 * sq
    closs_ref[0] = 0.25 * sq
    qloss_ref[0] = sq
    idx_ref[0] = idx


def kernel(z, embedding):
    z = z.astype(jnp.float32)
    b, c, h, w = z.shape
    hw = h * w
    n = embedding.shape[0]
    z3 = z.reshape(b, c, hw)

    out_shapes = (
        jax.ShapeDtypeStruct((b, c, hw), jnp.float32),
        jax.ShapeDtypeStruct((b, hw, c), jnp.float32),
        jax.ShapeDtypeStruct((b, hw, c), jnp.float32),
        jax.ShapeDtypeStruct((b, hw, c), jnp.float32),
        jax.ShapeDtypeStruct((b, 1, hw), jnp.int32),
    )
    zq, loss, closs, qloss, idx = pl.pallas_call(
        _vq_body,
        grid=(b,),
        in_specs=[
            pl.BlockSpec((1, c, hw), lambda i: (i, 0, 0)),
            pl.BlockSpec((n, c), lambda i: (0, 0)),
        ],
        out_specs=(
            pl.BlockSpec((1, c, hw), lambda i: (i, 0, 0)),
            pl.BlockSpec((1, hw, c), lambda i: (i, 0, 0)),
            pl.BlockSpec((1, hw, c), lambda i: (i, 0, 0)),
            pl.BlockSpec((1, hw, c), lambda i: (i, 0, 0)),
            pl.BlockSpec((1, 1, hw), lambda i: (i, 0, 0)),
        ),
        out_shape=out_shapes,
    )(z3, embedding)

    return (
        zq.reshape(b, c, h, w),
        loss.reshape(b, h, w, c),
        closs.reshape(b, h, w, c),
        qloss.reshape(b, h, w, c),
        idx.reshape(-1),
    )


# speedup vs baseline: 1.7723x; 1.0008x over previous
import jax
import jax.numpy as jnp
from jax.experimental import pallas as pl
from jax.experimental.pallas import tpu as pltpu


def _vq_body(z_ref, emb_ref, zq_ref, loss_ref, closs_ref, qloss_ref, idx_ref):
    E = emb_ref[...]
    Z = z_ref[0]
    e_sq = jnp.sum(E * E, axis=1, keepdims=True)
    z_sq = jnp.sum(Z * Z, axis=0, keepdims=True)
    mm = jax.lax.dot_general(
        E, Z, (((1,), (0,)), ((), ())),
        preferred_element_type=jnp.float32)
    d = (z_sq + e_sq) - 2.0 * mm
    dmin = jnp.min(d, axis=0, keepdims=True)
    iota = jax.lax.broadcasted_iota(jnp.int32, d.shape, 0)
    big = jnp.int32(d.shape[0])
    idx = jnp.min(jnp.where(d == dmin, iota, big), axis=0,
                  keepdims=True)
    onehot = (iota == idx).astype(jnp.float32)
    zq_t = jax.lax.dot_general(
        E, onehot, (((0,), (0,)), ((), ())),
        preferred_element_type=jnp.float32)
    zq_ref[0] = zq_t
    diff = zq_t - Z
    sq_t = diff * diff
    sq = sq_t.T
    loss_ref[0] = 1.25 * sq
    closs_ref[0] = 0.25 * sq
    qloss_ref[0] = sq
    idx_ref[0] = idx


def kernel(z, embedding):
    z = z.astype(jnp.float32)
    b, c, h, w = z.shape
    hw = h * w
    n = embedding.shape[0]
    z3 = z.reshape(b, c, hw)

    out_shapes = (
        jax.ShapeDtypeStruct((b, c, hw), jnp.float32),
        jax.ShapeDtypeStruct((b, hw, c), jnp.float32),
        jax.ShapeDtypeStruct((b, hw, c), jnp.float32),
        jax.ShapeDtypeStruct((b, hw, c), jnp.float32),
        jax.ShapeDtypeStruct((b, 1, hw), jnp.int32),
    )
    zq, loss, closs, qloss, idx = pl.pallas_call(
        _vq_body,
        grid=(b,),
        in_specs=[
            pl.BlockSpec((1, c, hw), lambda i: (i, 0, 0)),
            pl.BlockSpec((n, c), lambda i: (0, 0)),
        ],
        out_specs=(
            pl.BlockSpec((1, c, hw), lambda i: (i, 0, 0)),
            pl.BlockSpec((1, hw, c), lambda i: (i, 0, 0)),
            pl.BlockSpec((1, hw, c), lambda i: (i, 0, 0)),
            pl.BlockSpec((1, hw, c), lambda i: (i, 0, 0)),
            pl.BlockSpec((1, 1, hw), lambda i: (i, 0, 0)),
        ),
        out_shape=out_shapes,
        compiler_params=pltpu.CompilerParams(
            dimension_semantics=("parallel",)),
    )(z3, embedding)

    return (
        zq.reshape(b, c, h, w),
        loss.reshape(b, h, w, c),
        closs.reshape(b, h, w, c),
        qloss.reshape(b, h, w, c),
        idx.reshape(-1),
    )
